# baseline (speedup 1.0000x reference)
import functools

import jax
import jax.numpy as jnp
from jax import lax
from jax.experimental import pallas as pl
from jax.experimental.pallas import tpu as pltpu
from jax.experimental.pallas import tpu_sc as plsc

H, W, NPF = 32, 32, 384
LANES = 16
NREG = NPF // LANES
NC, NS = 2, 16
NCHUNK = 4
RPC = W // NCHUNK


@functools.partial(
    pl.kernel,
    out_type=jax.ShapeDtypeStruct((H * W, 2 * NPF), jnp.float32),
    mesh=plsc.VectorSubcoreMesh(core_axis_name="c", subcore_axis_name="s"),
    scratch_types=[
        pltpu.VMEM((W, 2 * NPF), jnp.float32),
        pltpu.VMEM((NPF,), jnp.float32),
        pltpu.SemaphoreType.DMA,
        [pltpu.SemaphoreType.DMA] * NCHUNK,
        [pltpu.SemaphoreType.DMA] * NCHUNK,
    ],
)
def _pos_embed_sc(row_hbm, colpad_hbm, out_hbm, buf, row_v, sem_row,
                  sems_in, sems_out):
    h = lax.axis_index("s") * NC + lax.axis_index("c")

    cps_in = []
    for c in range(NCHUNK):
        cp = pltpu.make_async_copy(
            colpad_hbm.at[pl.ds(c * RPC, RPC), :],
            buf.at[pl.ds(c * RPC, RPC), :],
            sems_in[c])
        cp.start()
        cps_in.append(cp)

    cp_row = pltpu.make_async_copy(row_hbm.at[h], row_v, sem_row)
    cp_row.start()
    cp_row.wait()
    regs = [row_v[pl.ds(LANES * i, LANES)] for i in range(NREG)]

    def fill_row(r, carry):
        for i in range(NREG):
            buf[r, pl.ds(NPF + LANES * i, LANES)] = regs[i]
        return carry

    cps_out = []
    for c in range(NCHUNK):
        cps_in[c].wait()
        lax.fori_loop(c * RPC, (c + 1) * RPC, fill_row, 0, unroll=4)
        cp = pltpu.make_async_copy(
            buf.at[pl.ds(c * RPC, RPC), :],
            out_hbm.at[pl.ds(h * W + c * RPC, RPC), :],
            sems_out[c])
        cp.start()
        cps_out.append(cp)
    for cp in cps_out:
        cp.wait()


def kernel(tensor, row_embed, col_embed):
    del tensor
    colpad = jnp.pad(col_embed, ((0, 0), (0, NPF)))
    return _pos_embed_sc(row_embed, colpad)
